# baseline (speedup 1.0000x reference)
import functools

import jax
import jax.numpy as jnp
from jax import lax
from jax.experimental import pallas as pl
from jax.experimental.pallas import tpu as pltpu
from jax.experimental.pallas import tpu_sc as plsc

N = 10000
E = 320000
F = 128
NC = 2
NS = 16
NW = NC * NS
NPAD = 10240
ST = NPAD // NS
C = 80
EPW = E // NW
KPW = EPW // C
EPT = E // NS
KPT = EPT // C


def _sc_body(row_h, col_h, ew_h, x_h, out_h,
             dis_v, degb, normv,
             rs0, rs1, rs2, cs0, cs1, cs2, ws0, ws1, ws2,
             cd0, cd1, cd2, cd3, wd0, wd1, wd2, wd3,
             xbuf0, xbuf1, xbuf2, dacc, tacc,
             deg_sh, dis_sh, s_sh,
             semg0, semg1, semg2, semp0, semp1, semp2,
             semd0, semd1, semd2, semd3, sempd0, sempd1, sempd2, sempd3):
    cid = lax.axis_index("c")
    sid = lax.axis_index("s")
    wid = cid * NS + sid
    xbufs = (xbuf0, xbuf1, xbuf2)
    semgs = (semg0, semg1, semg2)
    rss = (rs0, rs1, rs2)
    css = (cs0, cs1, cs2)
    wss = (ws0, ws1, ws2)
    semps = (semp0, semp1, semp2)
    cds = (cd0, cd1, cd2, cd3)
    wds = (wd0, wd1, wd2, wd3)
    semds = (semd0, semd1, semd2, semd3)
    sempds = (sempd0, sempd1, sempd2, sempd3)

    def zx(j, _):
        for f in range(F // 16):
            xbuf0[j, pl.ds(f * 16, 16)] = jnp.zeros((16,), jnp.float32)
        return 0
    lax.fori_loop(0, C, zx, 0)

    def zd(g, _):
        degb[pl.ds(g * 16, 16)] = jnp.zeros((16,), jnp.float32)
        return 0
    lax.fori_loop(0, ST // 16, zd, 0)

    for i in range(ST // C):
        pltpu.sync_copy(xbuf0, s_sh.at[pl.ds(sid * ST + i * C, C)])
    plsc.subcore_barrier()

    dbase = sid * EPT

    def dpre(k, s):
        pltpu.async_copy(col_h.at[pl.ds(dbase + k * C, C)], cds[s],
                         sempds[s])
        pltpu.async_copy(ew_h.at[pl.ds(dbase + k * C, C)], wds[s],
                         sempds[s])

    def dwait(k, s):
        pltpu.make_async_copy(col_h.at[pl.ds(dbase + k * C, C)], cds[s],
                              sempds[s]).wait()
        pltpu.make_async_copy(ew_h.at[pl.ds(dbase + k * C, C)], wds[s],
                              sempds[s]).wait()

    def dacc_local(s):
        for g in range(C // 16):
            c16 = cds[s][pl.ds(g * 16, 16)]
            w16 = wds[s][pl.ds(g * 16, 16)]
            hi = lax.shift_right_logical(c16, 7)
            lo = lax.bitwise_and(c16, jnp.int32(127))
            plsc.addupdate_scatter(xbuf0, [hi, lo], w16)

    dpre(0, 0)
    dpre(1, 1)
    dwait(0, 0)
    dpre(2, 2)
    dacc_local(0)
    dwait(1, 1)
    dpre(3, 3)
    dacc_local(1)

    def dstep(i, _):
        for j in range(4):
            k = 2 + 4 * i + j
            s = (2 + j) % 4
            dwait(k, s)

            @pl.when(k + 2 <= KPT - 1)
            def _pre():
                dpre(k + 2, j)
            dacc_local(s)
        return 0
    lax.fori_loop(0, (KPT - 2) // 4, dstep, 0)
    pltpu.sync_copy(xbuf0, s_sh.at[pl.ds(sid * (ST // 8), ST // 8)])
    plsc.subcore_barrier()

    def zacc(g, _):
        dacc[g // 8, pl.ds((g % 8) * 16, 16)] = jnp.zeros((16,), jnp.float32)
        return 0
    lax.fori_loop(0, ST // 16, zacc, 0)
    for l in range(NS):
        pltpu.sync_copy(
            s_sh.at[pl.ds(l * (ST // 8) + (ST // F) * sid, ST // F)], tacc)

        def racc(g, _):
            dacc[g // 8, pl.ds((g % 8) * 16, 16)] = (
                dacc[g // 8, pl.ds((g % 8) * 16, 16)]
                + tacc[g // 8, pl.ds((g % 8) * 16, 16)])
            return 0
        lax.fori_loop(0, ST // 16, racc, 0)

    def dis_step(g, _):
        d = dacc[g // 8, pl.ds((g % 8) * 16, 16)]
        i = lax.bitcast_convert_type(d, jnp.int32)
        i = jnp.int32(0x5F3759DF) - lax.shift_right_arithmetic(i, 1)
        y = lax.bitcast_convert_type(i, jnp.float32)
        for _ in range(3):
            y = y * (1.5 - 0.5 * d * y * y)
        degb[pl.ds(g * 16, 16)] = jnp.where(d > 0.0, y, 0.0)
        return 0
    lax.fori_loop(0, ST // 16, dis_step, 0)
    def zx2(j, _):
        for f in range(F // 16):
            xbuf0[j, pl.ds(f * 16, 16)] = jnp.zeros((16,), jnp.float32)
        return 0
    lax.fori_loop(0, C, zx2, 0)
    pltpu.sync_copy(xbuf0, s_sh.at[pl.ds(sid * (ST // 8), ST // 8)])
    pltpu.sync_copy(degb, dis_sh.at[pl.ds(sid * ST, ST)])
    plsc.subcore_barrier()
    pltpu.sync_copy(dis_sh, dis_v)

    base = wid * EPW

    def ipre(q, b):
        off = base + q * C
        pltpu.async_copy(row_h.at[pl.ds(off, C)], rss[b], semps[b])
        pltpu.async_copy(col_h.at[pl.ds(off, C)], css[b], semps[b])
        pltpu.async_copy(ew_h.at[pl.ds(off, C)], wss[b], semps[b])

    def iwait(q, b):
        off = base + q * C
        pltpu.make_async_copy(row_h.at[pl.ds(off, C)], rss[b],
                              semps[b]).wait()
        pltpu.make_async_copy(col_h.at[pl.ds(off, C)], css[b],
                              semps[b]).wait()
        pltpu.make_async_copy(ew_h.at[pl.ds(off, C)], wss[b],
                              semps[b]).wait()

    def process(q, b):
        for g in range(C // 16):
            r16 = rss[b][pl.ds(g * 16, 16)]
            c16 = css[b][pl.ds(g * 16, 16)]
            w16 = wss[b][pl.ds(g * 16, 16)]
            dr = plsc.load_gather(dis_v, [r16])
            dc = plsc.load_gather(dis_v, [c16])
            normv[pl.ds(g * 16, 16)] = dr * w16 * dc

        def scale(u, _):
            for t in range(2):
                jj = 2 * u + t
                nj = plsc.load_gather(normv, [jnp.full((16,), jj, jnp.int32)])
                for f in range(F // 16):
                    xbufs[b][jj, pl.ds(f * 16, 16)] = (
                        xbufs[b][jj, pl.ds(f * 16, 16)] * nj)
            return 0
        lax.fori_loop(0, C // 2, scale, 0)
        pltpu.sync_copy(xbufs[b], s_sh.at[css[b]], add=True)

    def T(q, b):
        pltpu.make_async_copy(x_h.at[rss[b]], xbufs[b], semgs[b]).wait()
        b2 = (b + 2) % 3

        @pl.when(q + 2 <= KPW - 1)
        def _g():
            iwait(q + 2, b2)
            pltpu.async_copy(x_h.at[rss[b2]], xbufs[b2], semgs[b2])
        process(q, b)

        @pl.when(q + 3 <= KPW - 1)
        def _p():
            ipre(q + 3, b)

    ipre(0, 0)
    ipre(1, 1)
    ipre(2, 2)
    iwait(0, 0)
    pltpu.async_copy(x_h.at[rs0], xbuf0, semg0)
    iwait(1, 1)
    pltpu.async_copy(x_h.at[rs1], xbuf1, semg1)
    T(0, 0)
    T(1, 1)

    def mbody(i, _):
        for j in range(3):
            q = 2 + 3 * i + j
            T(q, (2 + j) % 3)
        return 0
    lax.fori_loop(0, (KPW - 2) // 3, mbody, 0)
    plsc.subcore_barrier()

    pltpu.sync_copy(s_sh.at[pl.ds(sid * ST, ST)],
                    out_h.at[cid, pl.ds(sid * ST, ST)])


_sc_scatter = pl.kernel(
    _sc_body,
    out_type=jax.ShapeDtypeStruct((NC, NPAD, F), jnp.float32),
    mesh=plsc.VectorSubcoreMesh(core_axis_name="c", subcore_axis_name="s",
                                num_cores=NC, num_subcores=NS),
    scratch_types=[
        pltpu.VMEM((NPAD,), jnp.float32),
        pltpu.VMEM((ST,), jnp.float32),
        pltpu.VMEM((C,), jnp.float32),
        pltpu.VMEM((C,), jnp.int32),
        pltpu.VMEM((C,), jnp.int32),
        pltpu.VMEM((C,), jnp.int32),
        pltpu.VMEM((C,), jnp.int32),
        pltpu.VMEM((C,), jnp.int32),
        pltpu.VMEM((C,), jnp.int32),
        pltpu.VMEM((C,), jnp.float32),
        pltpu.VMEM((C,), jnp.float32),
        pltpu.VMEM((C,), jnp.float32),
        pltpu.VMEM((C,), jnp.int32),
        pltpu.VMEM((C,), jnp.int32),
        pltpu.VMEM((C,), jnp.int32),
        pltpu.VMEM((C,), jnp.int32),
        pltpu.VMEM((C,), jnp.float32),
        pltpu.VMEM((C,), jnp.float32),
        pltpu.VMEM((C,), jnp.float32),
        pltpu.VMEM((C,), jnp.float32),
        pltpu.VMEM((C, F), jnp.float32),
        pltpu.VMEM((C, F), jnp.float32),
        pltpu.VMEM((C, F), jnp.float32),
        pltpu.VMEM((ST // F, F), jnp.float32),
        pltpu.VMEM((ST // F, F), jnp.float32),
        pltpu.VMEM_SHARED((NPAD,), jnp.float32),
        pltpu.VMEM_SHARED((NPAD,), jnp.float32),
        pltpu.VMEM_SHARED((NPAD, F), jnp.float32),
        pltpu.SemaphoreType.DMA,
        pltpu.SemaphoreType.DMA,
        pltpu.SemaphoreType.DMA,
        pltpu.SemaphoreType.DMA,
        pltpu.SemaphoreType.DMA,
        pltpu.SemaphoreType.DMA,
        pltpu.SemaphoreType.DMA,
        pltpu.SemaphoreType.DMA,
        pltpu.SemaphoreType.DMA,
        pltpu.SemaphoreType.DMA,
        pltpu.SemaphoreType.DMA,
        pltpu.SemaphoreType.DMA,
        pltpu.SemaphoreType.DMA,
        pltpu.SemaphoreType.DMA,
    ],
    compiler_params=pltpu.CompilerParams(needs_layout_passes=False),
)


def _tc_body(x_ref, s0_ref, s1_ref, wl_ref, wg_ref, o_ref):
    dn = (((1,), (1,)), ((), ()))
    s = s0_ref[...] + s1_ref[...]
    o_ref[...] = (
        lax.dot_general(x_ref[...], wl_ref[...], dn,
                        preferred_element_type=jnp.float32,
                        precision=lax.Precision.HIGHEST)
        + lax.dot_general(s, wg_ref[...], dn,
                          preferred_element_type=jnp.float32,
                          precision=lax.Precision.HIGHEST))


_BLK = 1000


def _tc_combine(x, s0, s1, W_lin, W_gcn):
    grid = (N // _BLK,)
    row_spec = pl.BlockSpec((_BLK, F), lambda i: (i, 0))
    w_spec = pl.BlockSpec((F, F), lambda i: (0, 0))
    return pl.pallas_call(
        _tc_body,
        grid=grid,
        in_specs=[row_spec, row_spec, row_spec, w_spec, w_spec],
        out_specs=row_spec,
        out_shape=jax.ShapeDtypeStruct((N, F), jnp.float32),
    )(x, s0, s1, W_lin, W_gcn)


@jax.jit
def kernel(x, edge_index, edge_weight, W_lin, W_gcn):
    row = edge_index[0]
    col = edge_index[1]
    s_part = _sc_scatter(row, col, edge_weight, x)
    return _tc_combine(x, s_part[0, :N], s_part[1, :N], W_lin, W_gcn)


# speedup vs baseline: 1.1338x; 1.1338x over previous
import functools

import jax
import jax.numpy as jnp
from jax import lax
from jax.experimental import pallas as pl
from jax.experimental.pallas import tpu as pltpu
from jax.experimental.pallas import tpu_sc as plsc

N = 10000
E = 320000
F = 128
NC = 2
NS = 16
NW = NC * NS
NPAD = 10240
ST = NPAD // NS
C = 80
EPW = E // NW
KPW = EPW // C
EPT = E // NS
DC = 800
KD = EPT // DC


def _sc_body(row_h, col_h, ew_h, x_h, out_h,
             dis_v, degb, normv,
             rs0, rs1, rs2, cs0, cs1, cs2, ws0, ws1, ws2,
             cb0, cb1, wb0, wb1,
             xbuf0, xbuf1, xbuf2, dacc, tacc,
             dis_sh, s_sh,
             semg0, semg1, semg2, semp0, semp1, semp2,
             sempd0, sempd1):
    cid = lax.axis_index("c")
    sid = lax.axis_index("s")
    wid = cid * NS + sid
    xbufs = (xbuf0, xbuf1, xbuf2)
    semgs = (semg0, semg1, semg2)
    rss = (rs0, rs1, rs2)
    css = (cs0, cs1, cs2)
    wss = (ws0, ws1, ws2)
    semps = (semp0, semp1, semp2)
    cbs = (cb0, cb1)
    wbs = (wb0, wb1)
    sempds = (sempd0, sempd1)

    def zx(j, _):
        for f in range(F // 16):
            xbuf0[j, pl.ds(f * 16, 16)] = jnp.zeros((16,), jnp.float32)
        return 0
    lax.fori_loop(0, C, zx, 0)

    def zd(g, _):
        degb[pl.ds(g * 16, 16)] = jnp.zeros((16,), jnp.float32)
        return 0
    lax.fori_loop(0, ST // 16, zd, 0)

    for i in range(ST // C):
        pltpu.sync_copy(xbuf0, s_sh.at[pl.ds(sid * ST + i * C, C)])
    plsc.subcore_barrier()

    dbase = sid * EPT

    def dpre(k, p):
        off = dbase + k * DC
        pltpu.async_copy(col_h.at[pl.ds(off, DC)], cbs[p], sempds[p])
        pltpu.async_copy(ew_h.at[pl.ds(off, DC)], wbs[p], sempds[p])

    def dwait(k, p):
        off = dbase + k * DC
        pltpu.make_async_copy(col_h.at[pl.ds(off, DC)], cbs[p],
                              sempds[p]).wait()
        pltpu.make_async_copy(ew_h.at[pl.ds(off, DC)], wbs[p],
                              sempds[p]).wait()

    def dacc_local(p):
        def dg(g, _):
            c16 = cbs[p][pl.ds(g * 16, 16)]
            w16 = wbs[p][pl.ds(g * 16, 16)]
            hi = lax.shift_right_logical(c16, 7)
            lo = lax.bitwise_and(c16, jnp.int32(127))
            plsc.addupdate_scatter(xbuf0, [hi, lo], w16)
            return 0
        lax.fori_loop(0, DC // 16, dg, 0)

    dpre(0, 0)
    dwait(0, 0)
    dpre(1, 1)
    dacc_local(0)

    def dstep(i, _):
        for j in range(2):
            k = 1 + 2 * i + j
            p = (1 + j) % 2
            dwait(k, p)

            @pl.when(k + 1 <= KD - 1)
            def _pre():
                dpre(k + 1, 1 - p)
            dacc_local(p)
        return 0
    lax.fori_loop(0, (KD - 1) // 2, dstep, 0)
    pltpu.sync_copy(xbuf0, s_sh.at[pl.ds(sid * (ST // 8), ST // 8)])
    plsc.subcore_barrier()

    def zacc(g, _):
        dacc[g // 8, pl.ds((g % 8) * 16, 16)] = jnp.zeros((16,), jnp.float32)
        return 0
    lax.fori_loop(0, ST // 16, zacc, 0)
    for l in range(NS):
        pltpu.sync_copy(
            s_sh.at[pl.ds(l * (ST // 8) + (ST // F) * sid, ST // F)], tacc)

        def racc(g, _):
            dacc[g // 8, pl.ds((g % 8) * 16, 16)] = (
                dacc[g // 8, pl.ds((g % 8) * 16, 16)]
                + tacc[g // 8, pl.ds((g % 8) * 16, 16)])
            return 0
        lax.fori_loop(0, ST // 16, racc, 0)

    def dis_step(g, _):
        d = dacc[g // 8, pl.ds((g % 8) * 16, 16)]
        i = lax.bitcast_convert_type(d, jnp.int32)
        i = jnp.int32(0x5F3759DF) - lax.shift_right_arithmetic(i, 1)
        y = lax.bitcast_convert_type(i, jnp.float32)
        for _ in range(3):
            y = y * (1.5 - 0.5 * d * y * y)
        degb[pl.ds(g * 16, 16)] = jnp.where(d > 0.0, y, 0.0)
        return 0
    lax.fori_loop(0, ST // 16, dis_step, 0)
    def zx2(j, _):
        for f in range(F // 16):
            xbuf0[j, pl.ds(f * 16, 16)] = jnp.zeros((16,), jnp.float32)
        return 0
    lax.fori_loop(0, C, zx2, 0)
    pltpu.sync_copy(xbuf0, s_sh.at[pl.ds(sid * (ST // 8), ST // 8)])
    pltpu.sync_copy(degb, dis_sh.at[pl.ds(sid * ST, ST)])
    plsc.subcore_barrier()
    pltpu.sync_copy(dis_sh, dis_v)

    base = wid * EPW

    def ipre(q, b):
        off = base + q * C
        pltpu.async_copy(row_h.at[pl.ds(off, C)], rss[b], semps[b])
        pltpu.async_copy(col_h.at[pl.ds(off, C)], css[b], semps[b])
        pltpu.async_copy(ew_h.at[pl.ds(off, C)], wss[b], semps[b])

    def iwait(q, b):
        off = base + q * C
        pltpu.make_async_copy(row_h.at[pl.ds(off, C)], rss[b],
                              semps[b]).wait()
        pltpu.make_async_copy(col_h.at[pl.ds(off, C)], css[b],
                              semps[b]).wait()
        pltpu.make_async_copy(ew_h.at[pl.ds(off, C)], wss[b],
                              semps[b]).wait()

    def process(q, b):
        for g in range(C // 16):
            r16 = rss[b][pl.ds(g * 16, 16)]
            c16 = css[b][pl.ds(g * 16, 16)]
            w16 = wss[b][pl.ds(g * 16, 16)]
            dr = plsc.load_gather(dis_v, [r16])
            dc = plsc.load_gather(dis_v, [c16])
            normv[pl.ds(g * 16, 16)] = dr * w16 * dc

        def scale(u, _):
            for t in range(2):
                jj = 2 * u + t
                nj = plsc.load_gather(normv, [jnp.full((16,), jj, jnp.int32)])
                for f in range(F // 16):
                    xbufs[b][jj, pl.ds(f * 16, 16)] = (
                        xbufs[b][jj, pl.ds(f * 16, 16)] * nj)
            return 0
        lax.fori_loop(0, C // 2, scale, 0)
        pltpu.sync_copy(xbufs[b], s_sh.at[css[b]], add=True)

    def T(q, b):
        pltpu.make_async_copy(x_h.at[rss[b]], xbufs[b], semgs[b]).wait()
        b2 = (b + 2) % 3

        @pl.when(q + 2 <= KPW - 1)
        def _g():
            iwait(q + 2, b2)
            pltpu.async_copy(x_h.at[rss[b2]], xbufs[b2], semgs[b2])
        process(q, b)

        @pl.when(q + 3 <= KPW - 1)
        def _p():
            ipre(q + 3, b)

    ipre(0, 0)
    ipre(1, 1)
    ipre(2, 2)
    iwait(0, 0)
    pltpu.async_copy(x_h.at[rs0], xbuf0, semg0)
    iwait(1, 1)
    pltpu.async_copy(x_h.at[rs1], xbuf1, semg1)
    T(0, 0)
    T(1, 1)

    def mbody(i, _):
        for j in range(3):
            q = 2 + 3 * i + j
            T(q, (2 + j) % 3)
        return 0
    lax.fori_loop(0, (KPW - 2) // 3, mbody, 0)
    plsc.subcore_barrier()

    pltpu.sync_copy(s_sh.at[pl.ds(sid * ST, ST)],
                    out_h.at[cid, pl.ds(sid * ST, ST)])


_sc_scatter = pl.kernel(
    _sc_body,
    out_type=jax.ShapeDtypeStruct((NC, NPAD, F), jnp.float32),
    mesh=plsc.VectorSubcoreMesh(core_axis_name="c", subcore_axis_name="s",
                                num_cores=NC, num_subcores=NS),
    scratch_types=[
        pltpu.VMEM((NPAD,), jnp.float32),
        pltpu.VMEM((ST,), jnp.float32),
        pltpu.VMEM((C,), jnp.float32),
        pltpu.VMEM((C,), jnp.int32),
        pltpu.VMEM((C,), jnp.int32),
        pltpu.VMEM((C,), jnp.int32),
        pltpu.VMEM((C,), jnp.int32),
        pltpu.VMEM((C,), jnp.int32),
        pltpu.VMEM((C,), jnp.int32),
        pltpu.VMEM((C,), jnp.float32),
        pltpu.VMEM((C,), jnp.float32),
        pltpu.VMEM((C,), jnp.float32),
        pltpu.VMEM((DC,), jnp.int32),
        pltpu.VMEM((DC,), jnp.int32),
        pltpu.VMEM((DC,), jnp.float32),
        pltpu.VMEM((DC,), jnp.float32),
        pltpu.VMEM((C, F), jnp.float32),
        pltpu.VMEM((C, F), jnp.float32),
        pltpu.VMEM((C, F), jnp.float32),
        pltpu.VMEM((ST // F, F), jnp.float32),
        pltpu.VMEM((ST // F, F), jnp.float32),
        pltpu.VMEM_SHARED((NPAD,), jnp.float32),
        pltpu.VMEM_SHARED((NPAD, F), jnp.float32),
        pltpu.SemaphoreType.DMA,
        pltpu.SemaphoreType.DMA,
        pltpu.SemaphoreType.DMA,
        pltpu.SemaphoreType.DMA,
        pltpu.SemaphoreType.DMA,
        pltpu.SemaphoreType.DMA,
        pltpu.SemaphoreType.DMA,
        pltpu.SemaphoreType.DMA,
    ],
    compiler_params=pltpu.CompilerParams(needs_layout_passes=False),
)


def _tc_body(x_ref, s0_ref, s1_ref, wl_ref, wg_ref, o_ref):
    dn = (((1,), (1,)), ((), ()))
    s = s0_ref[...] + s1_ref[...]
    o_ref[...] = (
        lax.dot_general(x_ref[...], wl_ref[...], dn,
                        preferred_element_type=jnp.float32,
                        precision=lax.Precision.HIGHEST)
        + lax.dot_general(s, wg_ref[...], dn,
                          preferred_element_type=jnp.float32,
                          precision=lax.Precision.HIGHEST))


_BLK = 1000


def _tc_combine(x, s0, s1, W_lin, W_gcn):
    grid = (N // _BLK,)
    row_spec = pl.BlockSpec((_BLK, F), lambda i: (i, 0))
    w_spec = pl.BlockSpec((F, F), lambda i: (0, 0))
    return pl.pallas_call(
        _tc_body,
        grid=grid,
        in_specs=[row_spec, row_spec, row_spec, w_spec, w_spec],
        out_specs=row_spec,
        out_shape=jax.ShapeDtypeStruct((N, F), jnp.float32),
    )(x, s0, s1, W_lin, W_gcn)


@jax.jit
def kernel(x, edge_index, edge_weight, W_lin, W_gcn):
    row = edge_index[0]
    col = edge_index[1]
    s_part = _sc_scatter(row, col, edge_weight, x)
    return _tc_combine(x, s_part[0, :N], s_part[1, :N], W_lin, W_gcn)


# speedup vs baseline: 1.2928x; 1.1402x over previous
import functools

import jax
import jax.numpy as jnp
from jax import lax
from jax.experimental import pallas as pl
from jax.experimental.pallas import tpu as pltpu
from jax.experimental.pallas import tpu_sc as plsc

N = 10000
E = 320000
F = 128
NC = 2
NS = 16
NW = NC * NS
NPAD = 10240
ST = NPAD // NS
C = 80
EPW = E // NW
KPW = EPW // C
EPT = E // NS
DC = 800
KD = EPT // DC
RB = 2000
KR = RB // C


def _sc_body(row_h, col_h, ew_h, x_h, out_h,
             dis_v, degb, normv,
             rs0, rs1, cs0,
             cb0, cb1, wb0, wb1, rowb, colb, ewb,
             xbuf0, xbuf1, dacc, tacc,
             dis_sh, s_sh,
             semg0, semg1, sempd0, sempd1):
    cid = lax.axis_index("c")
    sid = lax.axis_index("s")
    wid = cid * NS + sid
    xbufs = (xbuf0, xbuf1)
    semgs = (semg0, semg1)
    rss = (rs0, rs1)
    cbs = (cb0, cb1)
    wbs = (wb0, wb1)
    sempds = (sempd0, sempd1)

    def zx(j, _):
        for f in range(F // 16):
            xbuf0[j, pl.ds(f * 16, 16)] = jnp.zeros((16,), jnp.float32)
        return 0
    lax.fori_loop(0, C, zx, 0)

    def zd(g, _):
        degb[pl.ds(g * 16, 16)] = jnp.zeros((16,), jnp.float32)
        return 0
    lax.fori_loop(0, ST // 16, zd, 0)

    for i in range(ST // C):
        pltpu.sync_copy(xbuf0, s_sh.at[pl.ds(sid * ST + i * C, C)])
    plsc.subcore_barrier()

    dbase = sid * EPT

    def dpre(k, p):
        off = dbase + k * DC
        pltpu.async_copy(col_h.at[pl.ds(off, DC)], cbs[p], sempds[p])
        pltpu.async_copy(ew_h.at[pl.ds(off, DC)], wbs[p], sempds[p])

    def dwait(k, p):
        off = dbase + k * DC
        pltpu.make_async_copy(col_h.at[pl.ds(off, DC)], cbs[p],
                              sempds[p]).wait()
        pltpu.make_async_copy(ew_h.at[pl.ds(off, DC)], wbs[p],
                              sempds[p]).wait()

    def dacc_local(p):
        def dg(g, _):
            c16 = cbs[p][pl.ds(g * 16, 16)]
            w16 = wbs[p][pl.ds(g * 16, 16)]
            hi = lax.shift_right_logical(c16, 7)
            lo = lax.bitwise_and(c16, jnp.int32(127))
            plsc.addupdate_scatter(xbuf0, [hi, lo], w16)
            return 0
        lax.fori_loop(0, DC // 16, dg, 0)

    dpre(0, 0)
    dwait(0, 0)
    dpre(1, 1)
    dacc_local(0)

    def dstep(i, _):
        for j in range(2):
            k = 1 + 2 * i + j
            p = (1 + j) % 2
            dwait(k, p)

            @pl.when(k + 1 <= KD - 1)
            def _pre():
                dpre(k + 1, 1 - p)
            dacc_local(p)
        return 0
    lax.fori_loop(0, (KD - 1) // 2, dstep, 0)
    pltpu.sync_copy(xbuf0, s_sh.at[pl.ds(sid * (ST // 8), ST // 8)])
    plsc.subcore_barrier()

    def zacc(g, _):
        dacc[g // 8, pl.ds((g % 8) * 16, 16)] = jnp.zeros((16,), jnp.float32)
        return 0
    lax.fori_loop(0, ST // 16, zacc, 0)
    for l in range(NS):
        pltpu.sync_copy(
            s_sh.at[pl.ds(l * (ST // 8) + (ST // F) * sid, ST // F)], tacc)

        def racc(g, _):
            dacc[g // 8, pl.ds((g % 8) * 16, 16)] = (
                dacc[g // 8, pl.ds((g % 8) * 16, 16)]
                + tacc[g // 8, pl.ds((g % 8) * 16, 16)])
            return 0
        lax.fori_loop(0, ST // 16, racc, 0)

    def dis_step(g, _):
        d = dacc[g // 8, pl.ds((g % 8) * 16, 16)]
        i = lax.bitcast_convert_type(d, jnp.int32)
        i = jnp.int32(0x5F3759DF) - lax.shift_right_arithmetic(i, 1)
        y = lax.bitcast_convert_type(i, jnp.float32)
        for _ in range(3):
            y = y * (1.5 - 0.5 * d * y * y)
        degb[pl.ds(g * 16, 16)] = jnp.where(d > 0.0, y, 0.0)
        return 0
    lax.fori_loop(0, ST // 16, dis_step, 0)
    def zx2(j, _):
        for f in range(F // 16):
            xbuf0[j, pl.ds(f * 16, 16)] = jnp.zeros((16,), jnp.float32)
        return 0
    lax.fori_loop(0, C, zx2, 0)
    pltpu.sync_copy(xbuf0, s_sh.at[pl.ds(sid * (ST // 8), ST // 8)])
    pltpu.sync_copy(degb, dis_sh.at[pl.ds(sid * ST, ST)])
    plsc.subcore_barrier()
    pltpu.sync_copy(dis_sh, dis_v)

    base = wid * EPW

    def stage(bulk, dst, k):
        for g in range(C // 16):
            dst[pl.ds(g * 16, 16)] = bulk[pl.ds(k * C + g * 16, 16)]

    def process(k, b):
        for g in range(C // 16):
            r16 = rowb[pl.ds(k * C + g * 16, 16)]
            c16 = colb[pl.ds(k * C + g * 16, 16)]
            w16 = ewb[pl.ds(k * C + g * 16, 16)]
            dr = plsc.load_gather(dis_v, [r16])
            dc = plsc.load_gather(dis_v, [c16])
            normv[pl.ds(g * 16, 16)] = dr * w16 * dc

        def scale(u, _):
            for t in range(2):
                jj = 2 * u + t
                nj = plsc.load_gather(normv, [jnp.full((16,), jj, jnp.int32)])
                for f in range(F // 16):
                    xbufs[b][jj, pl.ds(f * 16, 16)] = (
                        xbufs[b][jj, pl.ds(f * 16, 16)] * nj)
            return 0
        lax.fori_loop(0, C // 2, scale, 0)
        stage(colb, cs0, k)
        pltpu.sync_copy(xbufs[b], s_sh.at[cs0], add=True)

    def gstart(k, b):
        stage(rowb, rss[b], k)
        pltpu.async_copy(x_h.at[rss[b]], xbufs[b], semgs[b])

    def gwait(b):
        pltpu.make_async_copy(x_h.at[rss[b]], xbufs[b], semgs[b]).wait()

    def rnd(r, _):
        off = base + r * RB
        pltpu.sync_copy(row_h.at[pl.ds(off, RB)], rowb)
        pltpu.sync_copy(col_h.at[pl.ds(off, RB)], colb)
        pltpu.sync_copy(ew_h.at[pl.ds(off, RB)], ewb)
        gstart(0, 0)
        gstart(1, 1)
        gwait(0)
        process(0, 0)
        gstart(2, 0)

        def mstep(i, _):
            for j in range(2):
                k = 1 + 2 * i + j
                b = (1 + j) % 2
                gwait(b)
                process(k, b)

                @pl.when(k + 2 <= KR - 1)
                def _g():
                    gstart(k + 2, b)
            return 0
        lax.fori_loop(0, (KR - 1) // 2, mstep, 0)
        return 0
    lax.fori_loop(0, EPW // RB, rnd, 0)
    plsc.subcore_barrier()

    pltpu.sync_copy(s_sh.at[pl.ds(sid * ST, ST)],
                    out_h.at[cid, pl.ds(sid * ST, ST)])


_sc_scatter = pl.kernel(
    _sc_body,
    out_type=jax.ShapeDtypeStruct((NC, NPAD, F), jnp.float32),
    mesh=plsc.VectorSubcoreMesh(core_axis_name="c", subcore_axis_name="s",
                                num_cores=NC, num_subcores=NS),
    scratch_types=[
        pltpu.VMEM((NPAD,), jnp.float32),
        pltpu.VMEM((ST,), jnp.float32),
        pltpu.VMEM((C,), jnp.float32),
        pltpu.VMEM((C,), jnp.int32),
        pltpu.VMEM((C,), jnp.int32),
        pltpu.VMEM((C,), jnp.int32),
        pltpu.VMEM((DC,), jnp.int32),
        pltpu.VMEM((DC,), jnp.int32),
        pltpu.VMEM((DC,), jnp.float32),
        pltpu.VMEM((DC,), jnp.float32),
        pltpu.VMEM((RB,), jnp.int32),
        pltpu.VMEM((RB,), jnp.int32),
        pltpu.VMEM((RB,), jnp.float32),
        pltpu.VMEM((C, F), jnp.float32),
        pltpu.VMEM((C, F), jnp.float32),
        pltpu.VMEM((ST // F, F), jnp.float32),
        pltpu.VMEM((ST // F, F), jnp.float32),
        pltpu.VMEM_SHARED((NPAD,), jnp.float32),
        pltpu.VMEM_SHARED((NPAD, F), jnp.float32),
        pltpu.SemaphoreType.DMA,
        pltpu.SemaphoreType.DMA,
        pltpu.SemaphoreType.DMA,
        pltpu.SemaphoreType.DMA,
    ],
    compiler_params=pltpu.CompilerParams(needs_layout_passes=False),
)


def _tc_body(x_ref, s0_ref, s1_ref, wl_ref, wg_ref, o_ref):
    dn = (((1,), (1,)), ((), ()))
    s = s0_ref[...] + s1_ref[...]
    o_ref[...] = (
        lax.dot_general(x_ref[...], wl_ref[...], dn,
                        preferred_element_type=jnp.float32,
                        precision=lax.Precision.HIGHEST)
        + lax.dot_general(s, wg_ref[...], dn,
                          preferred_element_type=jnp.float32,
                          precision=lax.Precision.HIGHEST))


_BLK = 1000


def _tc_combine(x, s0, s1, W_lin, W_gcn):
    grid = (N // _BLK,)
    row_spec = pl.BlockSpec((_BLK, F), lambda i: (i, 0))
    w_spec = pl.BlockSpec((F, F), lambda i: (0, 0))
    return pl.pallas_call(
        _tc_body,
        grid=grid,
        in_specs=[row_spec, row_spec, row_spec, w_spec, w_spec],
        out_specs=row_spec,
        out_shape=jax.ShapeDtypeStruct((N, F), jnp.float32),
    )(x, s0, s1, W_lin, W_gcn)


@jax.jit
def kernel(x, edge_index, edge_weight, W_lin, W_gcn):
    row = edge_index[0]
    col = edge_index[1]
    s_part = _sc_scatter(row, col, edge_weight, x)
    return _tc_combine(x, s_part[0, :N], s_part[1, :N], W_lin, W_gcn)
